# baseline (speedup 1.0000x reference)
import jax
import jax.numpy as jnp
from jax.experimental import pallas as pl
from jax.experimental.pallas import tpu as pltpu

_HI = jax.lax.Precision.HIGHEST

_NBUF = 5
_PCHUNK = 480


def _make_kernel(n, f_in, h_dim, out_dim, c_dim, bm):
    nstripes = n // bm
    nsteps = 2 * nstripes
    ahead = _NBUF - 1

    def body(x_ref, adj_ref, fcw_ref, fcb_ref, w0_ref, b0_ref, w1_ref,
             b1_ref, wp_ref, bp_ref, out_ref, buf_ref, xb_ref, s0_ref,
             s1f_ref, s1_ref, sem, xsem):
        def issue(step, slot):
            stripe = jax.lax.rem(step, nstripes)
            pltpu.make_async_copy(
                adj_ref.at[pl.ds(stripe * bm, bm), :],
                buf_ref.at[slot],
                sem.at[slot],
            ).start()

        for j in range(ahead):
            issue(j, j)

        wa = jnp.dot(fcw_ref[...], w0_ref[...],
                     preferred_element_type=jnp.float32, precision=_HI)
        c = jnp.dot(fcb_ref[...], w0_ref[...],
                    preferred_element_type=jnp.float32, precision=_HI)
        wh = wa.astype(jnp.bfloat16)
        wl = (wa - wh.astype(jnp.float32)).astype(jnp.bfloat16)
        chunks = []
        r = 0
        while r < n:
            m = min(_PCHUNK, n - r)
            chunks.append((r, m))
            r += m

        def xcopy(idx):
            rr, mm = chunks[idx]
            return pltpu.make_async_copy(
                x_ref.at[pl.ds(rr, mm), :],
                xb_ref.at[idx % 2, pl.ds(0, mm), :],
                xsem.at[idx % 2],
            )

        xcopy(0).start()
        for idx, (rr, mm) in enumerate(chunks):
            if idx + 1 < len(chunks):
                xcopy(idx + 1).start()
            xcopy(idx).wait()
            x = xb_ref[idx % 2, :mm, :]
            xh = x.astype(jnp.bfloat16)
            xl = (x - xh.astype(jnp.float32)).astype(jnp.bfloat16)
            s0 = (jnp.dot(xh, wh, preferred_element_type=jnp.float32)
                  + jnp.dot(xh, wl, preferred_element_type=jnp.float32)
                  + jnp.dot(xl, wh, preferred_element_type=jnp.float32)) + c
            s0_ref[rr:rr + mm, :] = s0.astype(jnp.bfloat16)

        def step_fn(h, _):
            slot = jax.lax.rem(h, _NBUF)
            stripe = jax.lax.rem(h, nstripes)

            @pl.when(h + ahead < nsteps)
            def _issue_next():
                issue(h + ahead, jax.lax.rem(h + ahead, _NBUF))

            pltpu.make_async_copy(
                adj_ref.at[pl.ds(stripe * bm, bm), :],
                buf_ref.at[slot],
                sem.at[slot],
            ).wait()

            a = buf_ref[slot].astype(jnp.bfloat16)

            @pl.when(h < nstripes)
            def _pass_a():
                t = jnp.dot(a, s0_ref[...], preferred_element_type=jnp.float32)
                hh = jnp.maximum(t + b0_ref[...], 0.0)
                s1 = jnp.dot(hh, w1_ref[...],
                             preferred_element_type=jnp.float32, precision=_HI)
                s1f_ref[pl.ds(stripe * bm, bm), :] = s1

            @pl.when(h == nstripes)
            def _stage_s1():
                s1_ref[...] = s1f_ref[...].astype(jnp.bfloat16)

            @pl.when(h >= nstripes)
            def _pass_b():
                t = jnp.dot(a, s1_ref[...], preferred_element_type=jnp.float32)
                hh = jnp.maximum(t + b1_ref[...], 0.0)
                out_ref[pl.ds(stripe * bm, bm), :] = jnp.dot(
                    hh, wp_ref[...], preferred_element_type=jnp.float32,
                    precision=_HI) + bp_ref[...]

            return 0

        jax.lax.fori_loop(0, nsteps, step_fn, 0)

    return body


def _pick_block(n):
    for bm in (200, 400, 100, 80, 40, 25, 20, 16, 10, 8, 5, 4, 2, 1):
        if n % bm == 0 and n // bm >= _NBUF:
            return bm
    return n


def kernel(X, adj, fc_W, fc_b, conv0_W, conv0_b, conv1_W, conv1_b, pred_W, pred_b):
    x0 = X[0]
    n, f_in = x0.shape
    h_dim = conv0_W.shape[1]
    out_dim = conv1_W.shape[1]
    c_dim = pred_W.shape[1]
    bm = _pick_block(n)

    vmem = pl.BlockSpec(memory_space=pltpu.MemorySpace.VMEM)

    out = pl.pallas_call(
        _make_kernel(n, f_in, h_dim, out_dim, c_dim, bm),
        in_specs=[
            pl.BlockSpec(memory_space=pl.ANY),
            pl.BlockSpec(memory_space=pl.ANY),
            vmem, vmem, vmem, vmem, vmem, vmem, vmem, vmem,
        ],
        out_specs=vmem,
        out_shape=jax.ShapeDtypeStruct((n, c_dim), jnp.float32),
        scratch_shapes=[
            pltpu.VMEM((_NBUF, bm, n), jnp.float32),
            pltpu.VMEM((2, min(_PCHUNK, n), f_in), jnp.float32),
            pltpu.VMEM((n, h_dim), jnp.bfloat16),
            pltpu.VMEM((n, out_dim), jnp.float32),
            pltpu.VMEM((n, out_dim), jnp.bfloat16),
            pltpu.SemaphoreType.DMA((_NBUF,)),
            pltpu.SemaphoreType.DMA((2,)),
        ],
    )(x0, adj, fc_W, fc_b.reshape(1, -1), conv0_W, conv0_b.reshape(1, -1),
      conv1_W, conv1_b.reshape(1, -1), pred_W, pred_b.reshape(1, -1))

    return out


# speedup vs baseline: 1.0507x; 1.0507x over previous
import jax
import jax.numpy as jnp
from jax.experimental import pallas as pl
from jax.experimental.pallas import tpu as pltpu

_HI = jax.lax.Precision.HIGHEST

_NBUF = 4


def _make_kernel(n, f_in, h_dim, out_dim, c_dim, bm):
    nstripes = n // bm
    nsteps = 2 * nstripes
    ahead = _NBUF - 1

    def body(x_ref, adj_ref, fcw_ref, fcb_ref, w0_ref, b0_ref, w1_ref,
             b1_ref, wp_ref, bp_ref, out_ref, buf_ref, s0_ref,
             s1f_ref, s1_ref, sem):
        def issue(step, slot):
            stripe = jax.lax.rem(step, nstripes)
            pltpu.make_async_copy(
                adj_ref.at[pl.ds(stripe * bm, bm), :],
                buf_ref.at[slot],
                sem.at[slot],
            ).start()

        for j in range(ahead):
            issue(j, j)

        wa = jnp.dot(fcw_ref[...], w0_ref[...],
                     preferred_element_type=jnp.float32, precision=_HI)
        c = jnp.dot(fcb_ref[...], w0_ref[...],
                    preferred_element_type=jnp.float32, precision=_HI)
        wh = wa.astype(jnp.bfloat16)
        wl = (wa - wh.astype(jnp.float32)).astype(jnp.bfloat16)
        pchunk = 2000 if n % 2000 == 0 else n
        r = 0
        while r < n:
            m = min(pchunk, n - r)
            x = x_ref[r:r + m, :]
            xh = x.astype(jnp.bfloat16)
            xl = (x - xh.astype(jnp.float32)).astype(jnp.bfloat16)
            s0 = (jnp.dot(xh, wh, preferred_element_type=jnp.float32)
                  + jnp.dot(xh, wl, preferred_element_type=jnp.float32)
                  + jnp.dot(xl, wh, preferred_element_type=jnp.float32)) + c
            s0_ref[r:r + m, :] = s0.astype(jnp.bfloat16)
            r += m

        def step_fn(h, _):
            slot = jax.lax.rem(h, _NBUF)
            stripe = jax.lax.rem(h, nstripes)

            pltpu.make_async_copy(
                adj_ref.at[pl.ds(stripe * bm, bm), :],
                buf_ref.at[slot],
                sem.at[slot],
            ).wait()

            @pl.when(h + ahead < nsteps)
            def _issue_next():
                issue(h + ahead, jax.lax.rem(h + ahead, _NBUF))

            a = buf_ref[slot].astype(jnp.bfloat16)

            @pl.when(h < nstripes)
            def _pass_a():
                t = jnp.dot(a, s0_ref[...], preferred_element_type=jnp.float32)
                hh = jnp.maximum(t + b0_ref[...], 0.0)
                s1 = jnp.dot(hh, w1_ref[...],
                             preferred_element_type=jnp.float32, precision=_HI)
                s1f_ref[pl.ds(stripe * bm, bm), :] = s1

            @pl.when(h == nstripes)
            def _stage_s1():
                s1_ref[...] = s1f_ref[...].astype(jnp.bfloat16)

            @pl.when(h >= nstripes)
            def _pass_b():
                t = jnp.dot(a, s1_ref[...], preferred_element_type=jnp.float32)
                hh = jnp.maximum(t + b1_ref[...], 0.0)
                out_ref[pl.ds(stripe * bm, bm), :] = jnp.dot(
                    hh, wp_ref[...], preferred_element_type=jnp.float32,
                    precision=_HI) + bp_ref[...]

            return 0

        jax.lax.fori_loop(0, nsteps, step_fn, 0)

    return body


def _pick_block(n):
    for bm in (200, 400, 100, 80, 40, 25, 20, 16, 10, 8, 5, 4, 2, 1):
        if n % bm == 0 and n // bm >= _NBUF:
            return bm
    return n


def kernel(X, adj, fc_W, fc_b, conv0_W, conv0_b, conv1_W, conv1_b, pred_W, pred_b):
    x0 = X[0]
    n, f_in = x0.shape
    h_dim = conv0_W.shape[1]
    out_dim = conv1_W.shape[1]
    c_dim = pred_W.shape[1]
    bm = _pick_block(n)

    vmem = pl.BlockSpec(memory_space=pltpu.MemorySpace.VMEM)

    out = pl.pallas_call(
        _make_kernel(n, f_in, h_dim, out_dim, c_dim, bm),
        in_specs=[
            vmem,
            pl.BlockSpec(memory_space=pl.ANY),
            vmem, vmem, vmem, vmem, vmem, vmem, vmem, vmem,
        ],
        out_specs=vmem,
        out_shape=jax.ShapeDtypeStruct((n, c_dim), jnp.float32),
        scratch_shapes=[
            pltpu.VMEM((_NBUF, bm, n), jnp.float32),
            pltpu.VMEM((n, h_dim), jnp.bfloat16),
            pltpu.VMEM((n, out_dim), jnp.float32),
            pltpu.VMEM((n, out_dim), jnp.bfloat16),
            pltpu.SemaphoreType.DMA((_NBUF,)),
        ],
    )(x0, adj, fc_W, fc_b.reshape(1, -1), conv0_W, conv0_b.reshape(1, -1),
      conv1_W, conv1_b.reshape(1, -1), pred_W, pred_b.reshape(1, -1))

    return out


# speedup vs baseline: 1.0567x; 1.0058x over previous
import jax
import jax.numpy as jnp
from jax.experimental import pallas as pl
from jax.experimental.pallas import tpu as pltpu

_HI = jax.lax.Precision.HIGHEST

_NBUF = 4


def _make_kernel(n, f_in, h_dim, out_dim, c_dim, bm):
    nstripes = n // bm
    nsteps = 2 * nstripes
    ahead = _NBUF - 1

    def body(x_ref, adj_ref, fcw_ref, fcb_ref, w0_ref, b0_ref, w1_ref,
             b1_ref, wp_ref, bp_ref, out_ref, buf_ref, s0_ref,
             s1f_ref, s1_ref, sem):
        def issue(step, slot):
            stripe = jax.lax.rem(step, nstripes)
            pltpu.make_async_copy(
                adj_ref.at[pl.ds(stripe * bm, bm), :],
                buf_ref.at[slot],
                sem.at[slot],
            ).start()

        for j in range(ahead):
            issue(j, j)

        wa = jnp.dot(fcw_ref[...], w0_ref[...],
                     preferred_element_type=jnp.float32, precision=_HI)
        c = jnp.dot(fcb_ref[...], w0_ref[...],
                    preferred_element_type=jnp.float32, precision=_HI)
        wh = wa.astype(jnp.bfloat16)
        wl = (wa - wh.astype(jnp.float32)).astype(jnp.bfloat16)
        pchunk = 2000 if n % 2000 == 0 else n
        r = 0
        while r < n:
            m = min(pchunk, n - r)
            x = x_ref[r:r + m, :]
            xh = x.astype(jnp.bfloat16)
            xl = (x - xh.astype(jnp.float32)).astype(jnp.bfloat16)
            s0 = (jnp.dot(xh, wh, preferred_element_type=jnp.float32)
                  + jnp.dot(xh, wl, preferred_element_type=jnp.float32)
                  + jnp.dot(xl, wh, preferred_element_type=jnp.float32)) + c
            s0_ref[r:r + m, :] = s0.astype(jnp.bfloat16)
            r += m

        def step(h, slot, next_slot):
            stripe = jax.lax.rem(h, nstripes)

            pltpu.make_async_copy(
                adj_ref.at[pl.ds(stripe * bm, bm), :],
                buf_ref.at[slot],
                sem.at[slot],
            ).wait()

            @pl.when(h + ahead < nsteps)
            def _issue_next():
                issue(h + ahead, next_slot)

            a = buf_ref[slot].astype(jnp.bfloat16)

            @pl.when(h < nstripes)
            def _pass_a():
                t = jnp.dot(a, s0_ref[...], preferred_element_type=jnp.float32)
                hh = jnp.maximum(t + b0_ref[...], 0.0)
                s1 = jnp.dot(hh, w1_ref[...],
                             preferred_element_type=jnp.float32, precision=_HI)
                s1f_ref[pl.ds(stripe * bm, bm), :] = s1

            @pl.when(h == nstripes)
            def _stage_s1():
                s1_ref[...] = s1f_ref[...].astype(jnp.bfloat16)

            @pl.when(h >= nstripes)
            def _pass_b():
                t = jnp.dot(a, s1_ref[...], preferred_element_type=jnp.float32)
                hh = jnp.maximum(t + b1_ref[...], 0.0)
                out_ref[pl.ds(stripe * bm, bm), :] = jnp.dot(
                    hh, wp_ref[...], preferred_element_type=jnp.float32,
                    precision=_HI) + bp_ref[...]

        if nsteps % _NBUF == 0:
            def outer_fn(o, _):
                h0 = o * _NBUF
                for j in range(_NBUF):
                    step(h0 + j, j, (j + ahead) % _NBUF)
                return 0

            jax.lax.fori_loop(0, nsteps // _NBUF, outer_fn, 0)
        else:
            def step_fn(h, _):
                slot = jax.lax.rem(h, _NBUF)
                stripe0 = jax.lax.rem(h, nstripes)
                pltpu.make_async_copy(
                    adj_ref.at[pl.ds(stripe0 * bm, bm), :],
                    buf_ref.at[slot],
                    sem.at[slot],
                ).wait()

                @pl.when(h + ahead < nsteps)
                def _issue_next():
                    issue(h + ahead, jax.lax.rem(h + ahead, _NBUF))

                a = buf_ref[slot].astype(jnp.bfloat16)

                @pl.when(h < nstripes)
                def _pass_a():
                    t = jnp.dot(a, s0_ref[...],
                                preferred_element_type=jnp.float32)
                    hh = jnp.maximum(t + b0_ref[...], 0.0)
                    s1 = jnp.dot(hh, w1_ref[...],
                                 preferred_element_type=jnp.float32,
                                 precision=_HI)
                    s1f_ref[pl.ds(stripe0 * bm, bm), :] = s1

                @pl.when(h == nstripes)
                def _stage_s1():
                    s1_ref[...] = s1f_ref[...].astype(jnp.bfloat16)

                @pl.when(h >= nstripes)
                def _pass_b():
                    t = jnp.dot(a, s1_ref[...],
                                preferred_element_type=jnp.float32)
                    hh = jnp.maximum(t + b1_ref[...], 0.0)
                    out_ref[pl.ds(stripe0 * bm, bm), :] = jnp.dot(
                        hh, wp_ref[...], preferred_element_type=jnp.float32,
                        precision=_HI) + bp_ref[...]

                return 0

            jax.lax.fori_loop(0, nsteps, step_fn, 0)

    return body


def _pick_block(n):
    for bm in (200, 400, 100, 80, 40, 25, 20, 16, 10, 8, 5, 4, 2, 1):
        if n % bm == 0 and n // bm >= _NBUF:
            return bm
    return n


def kernel(X, adj, fc_W, fc_b, conv0_W, conv0_b, conv1_W, conv1_b, pred_W, pred_b):
    x0 = X[0]
    n, f_in = x0.shape
    h_dim = conv0_W.shape[1]
    out_dim = conv1_W.shape[1]
    c_dim = pred_W.shape[1]
    bm = _pick_block(n)

    vmem = pl.BlockSpec(memory_space=pltpu.MemorySpace.VMEM)

    out = pl.pallas_call(
        _make_kernel(n, f_in, h_dim, out_dim, c_dim, bm),
        in_specs=[
            vmem,
            pl.BlockSpec(memory_space=pl.ANY),
            vmem, vmem, vmem, vmem, vmem, vmem, vmem, vmem,
        ],
        out_specs=vmem,
        out_shape=jax.ShapeDtypeStruct((n, c_dim), jnp.float32),
        scratch_shapes=[
            pltpu.VMEM((_NBUF, bm, n), jnp.float32),
            pltpu.VMEM((n, h_dim), jnp.bfloat16),
            pltpu.VMEM((n, out_dim), jnp.float32),
            pltpu.VMEM((n, out_dim), jnp.bfloat16),
            pltpu.SemaphoreType.DMA((_NBUF,)),
        ],
    )(x0, adj, fc_W, fc_b.reshape(1, -1), conv0_W, conv0_b.reshape(1, -1),
      conv1_W, conv1_b.reshape(1, -1), pred_W, pred_b.reshape(1, -1))

    return out
